# baseline (speedup 1.0000x reference)
import functools

import jax
import jax.numpy as jnp
from jax import lax
from jax.experimental import pallas as pl
from jax.experimental.pallas import tpu as pltpu
from jax.experimental.pallas import tpu_sc as plsc

_R, _C = 16384, 4096
_N = _R * _C
_NW = 32
_WROWS = _R // _NW
_CR = 4
_CHUNK = _CR * _C
_Q = _CHUNK // 4
_NCHUNK = _WROWS // _CR
_SHIFT = 12


def _sc_gather(x1d, idx2):
    mesh = plsc.VectorSubcoreMesh(core_axis_name="c", subcore_axis_name="s")

    @functools.partial(
        pl.kernel,
        mesh=mesh,
        out_type=jax.ShapeDtypeStruct((_R, _C), jnp.float32),
        scratch_types=[
            pltpu.VMEM((_CHUNK,), jnp.int32),
            pltpu.VMEM((_CHUNK,), jnp.int32),
        ] + [pltpu.VMEM((_Q,), jnp.int32)] * 8
          + [pltpu.VMEM((_Q,), jnp.float32)] * 8
          + [pltpu.SemaphoreType.DMA] * 10,
    )
    def k(x_hbm, idx_hbm, out_hbm, idx_a, idx_b,
          fa0, fa1, fa2, fa3, fb0, fb1, fb2, fb3,
          da0, da1, da2, da3, db0, db1, db2, db3,
          sem_in, sem_out,
          sga0, sga1, sga2, sga3, sgb0, sgb1, sgb2, sgb3):
        wid = lax.axis_index("s") * 2 + lax.axis_index("c")
        base = wid * _WROWS
        lane = lax.iota(jnp.int32, 16)

        def idx_start(c, idx_v):
            for r in range(_CR):
                pltpu.make_async_copy(
                    idx_hbm.at[base + c * _CR + r],
                    idx_v.at[pl.ds(r * _C, _C)], sem_in).start()

        def idx_wait(c, idx_v):
            for r in range(_CR):
                pltpu.make_async_copy(
                    idx_hbm.at[base + c * _CR + r],
                    idx_v.at[pl.ds(r * _C, _C)], sem_in).wait()

        def fidx_compute(idx_v, fx):
            def frow(r, carry):
                col = (r << 4) + lane
                for q in range(4):
                    fx[q][pl.ds(r * 16, 16)] = (
                        (idx_v[pl.ds(q * _Q + r * 16, 16)] << _SHIFT) | col)
                return carry
            lax.fori_loop(0, _Q // 16, frow, 0, unroll=8)

        def gather_start(fidx_v, data_v, sem):
            pltpu.make_async_copy(x_hbm.at[fidx_v], data_v, sem).start()

        def gather_wait(fidx_v, data_v, sem):
            pltpu.make_async_copy(x_hbm.at[fidx_v], data_v, sem).wait()

        def out_start(c, dx):
            for r in range(_CR):
                pltpu.make_async_copy(
                    dx[r], out_hbm.at[base + c * _CR + r], sem_out).start()

        def out_wait(c, dx):
            for r in range(_CR):
                pltpu.make_async_copy(
                    dx[r], out_hbm.at[base + c * _CR + r], sem_out).wait()

        fxa = (fa0, fa1, fa2, fa3)
        fxb = (fb0, fb1, fb2, fb3)
        dxa = (da0, da1, da2, da3)
        dxb = (db0, db1, db2, db3)
        sxa = (sga0, sga1, sga2, sga3)
        sxb = (sgb0, sgb1, sgb2, sgb3)

        idx_start(0, idx_a)
        idx_start(1, idx_b)
        idx_wait(0, idx_a)
        fidx_compute(idx_a, fxa)
        for q in range(4):
            gather_start(fxa[q], dxa[q], sxa[q])

        def half(c, cur, nxt):
            (idx_c, fx_c, dx_c, sx_c) = cur
            (idx_n, fx_n, dx_n, sx_n) = nxt

            @pl.when(c + 1 < _NCHUNK)
            def _stage_next():
                idx_wait(c + 1, idx_n)
                fidx_compute(idx_n, fx_n)

            @pl.when(c > 0)
            def _drain_prev_out():
                out_wait(c - 1, dx_n)

            @pl.when(c + 1 < _NCHUNK)
            def _fire_next():
                for q in range(4):
                    gather_start(fx_n[q], dx_n[q], sx_n[q])

            for q in range(4):
                gather_wait(fx_c[q], dx_c[q], sx_c[q])
            out_start(c, dx_c)

            @pl.when(c + 2 < _NCHUNK)
            def _prefetch():
                idx_start(c + 2, idx_c)

        bufs_a = (idx_a, fxa, dxa, sxa)
        bufs_b = (idx_b, fxb, dxb, sxb)

        def pair_body(cp, carry):
            half(2 * cp, bufs_a, bufs_b)
            half(2 * cp + 1, bufs_b, bufs_a)
            return carry

        lax.fori_loop(0, _NCHUNK // 2, pair_body, 0)
        out_wait(_NCHUNK - 1, dxb)

    return k(x1d, idx2)


def kernel(x, index):
    x1d = x.reshape(_N)
    return _sc_gather(x1d, index)


# speedup vs baseline: 1.0958x; 1.0958x over previous
import functools

import jax
import jax.numpy as jnp
from jax import lax
from jax.experimental import pallas as pl
from jax.experimental.pallas import tpu as pltpu
from jax.experimental.pallas import tpu_sc as plsc

_R, _C = 16384, 4096
_N = _R * _C
_NW = 32
_WROWS = _R // _NW
_CR = 4
_CHUNK = _CR * _C
_HALF = _CHUNK // 2
_NCHUNK = _WROWS // _CR
_SHIFT = 12


def _sc_gather(x1d, idx2):
    mesh = plsc.VectorSubcoreMesh(core_axis_name="c", subcore_axis_name="s")

    @functools.partial(
        pl.kernel,
        mesh=mesh,
        out_type=jax.ShapeDtypeStruct((_R, _C), jnp.float32),
        scratch_types=[
            pltpu.VMEM((_CHUNK,), jnp.int32),
            pltpu.VMEM((_CHUNK,), jnp.int32),
            pltpu.VMEM((_HALF,), jnp.int32),
            pltpu.VMEM((_HALF,), jnp.int32),
            pltpu.VMEM((_HALF,), jnp.int32),
            pltpu.VMEM((_HALF,), jnp.int32),
            pltpu.VMEM((_HALF,), jnp.float32),
            pltpu.VMEM((_HALF,), jnp.float32),
            pltpu.VMEM((_HALF,), jnp.float32),
            pltpu.VMEM((_HALF,), jnp.float32),
            pltpu.SemaphoreType.DMA,
            pltpu.SemaphoreType.DMA,
            pltpu.SemaphoreType.DMA,
            pltpu.SemaphoreType.DMA,
            pltpu.SemaphoreType.DMA,
            pltpu.SemaphoreType.DMA,
        ],
    )
    def k(x_hbm, idx_hbm, out_hbm, idx_a, idx_b,
          fidx_a1, fidx_a2, fidx_b1, fidx_b2,
          data_a1, data_a2, data_b1, data_b2,
          sem_in, sem_out, sem_ga1, sem_ga2, sem_gb1, sem_gb2):
        wid = lax.axis_index("s") * 2 + lax.axis_index("c")
        base = wid * _WROWS
        lane = lax.iota(jnp.int32, 16)

        def idx_start(c, idx_v):
            for r in range(_CR):
                pltpu.make_async_copy(
                    idx_hbm.at[base + c * _CR + r],
                    idx_v.at[pl.ds(r * _C, _C)], sem_in).start()

        def idx_wait(c, idx_v):
            for r in range(_CR):
                pltpu.make_async_copy(
                    idx_hbm.at[base + c * _CR + r],
                    idx_v.at[pl.ds(r * _C, _C)], sem_in).wait()

        def fidx_compute(idx_v, fidx_1, fidx_2):
            def frow(r, carry):
                col = (lax.rem(r, _C // 16) << 4) + lane
                fidx_1[pl.ds(r * 16, 16)] = (
                    (idx_v[pl.ds(r * 16, 16)] << _SHIFT) | col)
                fidx_2[pl.ds(r * 16, 16)] = (
                    (idx_v[pl.ds(_HALF + r * 16, 16)] << _SHIFT) | col)
                return carry
            lax.fori_loop(0, _HALF // 16, frow, 0, unroll=8)

        def gather_start(fidx_v, data_v, sem):
            pltpu.make_async_copy(x_hbm.at[fidx_v], data_v, sem).start()

        def gather_wait(fidx_v, data_v, sem):
            pltpu.make_async_copy(x_hbm.at[fidx_v], data_v, sem).wait()

        def out_start(c, data_1, data_2):
            for r in range(_CR):
                d = data_1 if r < _CR // 2 else data_2
                o = (r % (_CR // 2)) * _C
                pltpu.make_async_copy(
                    d.at[pl.ds(o, _C)],
                    out_hbm.at[base + c * _CR + r], sem_out).start()

        def out_wait(c, data_1, data_2):
            for r in range(_CR):
                d = data_1 if r < _CR // 2 else data_2
                o = (r % (_CR // 2)) * _C
                pltpu.make_async_copy(
                    d.at[pl.ds(o, _C)],
                    out_hbm.at[base + c * _CR + r], sem_out).wait()

        idx_start(0, idx_a)
        idx_start(1, idx_b)
        idx_wait(0, idx_a)
        fidx_compute(idx_a, fidx_a1, fidx_a2)
        gather_start(fidx_a1, data_a1, sem_ga1)
        gather_start(fidx_a2, data_a2, sem_ga2)

        def half(c, cur, nxt):
            (idx_c, fidx_c1, fidx_c2, data_c1, data_c2, sem_c1, sem_c2) = cur
            (idx_n, fidx_n1, fidx_n2, data_n1, data_n2, sem_n1, sem_n2) = nxt

            @pl.when(c + 1 < _NCHUNK)
            def _stage_next():
                idx_wait(c + 1, idx_n)
                fidx_compute(idx_n, fidx_n1, fidx_n2)

            @pl.when(c > 0)
            def _drain_prev_out():
                out_wait(c - 1, data_n1, data_n2)

            @pl.when(c + 1 < _NCHUNK)
            def _fire_next():
                gather_start(fidx_n1, data_n1, sem_n1)
                gather_start(fidx_n2, data_n2, sem_n2)

            gather_wait(fidx_c1, data_c1, sem_c1)
            gather_wait(fidx_c2, data_c2, sem_c2)
            out_start(c, data_c1, data_c2)

            @pl.when(c + 2 < _NCHUNK)
            def _prefetch():
                idx_start(c + 2, idx_c)

        bufs_a = (idx_a, fidx_a1, fidx_a2, data_a1, data_a2, sem_ga1, sem_ga2)
        bufs_b = (idx_b, fidx_b1, fidx_b2, data_b1, data_b2, sem_gb1, sem_gb2)

        def pair_body(cp, carry):
            half(2 * cp, bufs_a, bufs_b)
            half(2 * cp + 1, bufs_b, bufs_a)
            return carry

        lax.fori_loop(0, _NCHUNK // 2, pair_body, 0)
        out_wait(_NCHUNK - 1, data_b1, data_b2)

    return k(x1d, idx2)


def kernel(x, index):
    x1d = x.reshape(_N)
    return _sc_gather(x1d, index)
